# baseline (speedup 1.0000x reference)
import functools

import jax
import jax.numpy as jnp
from jax import lax
from jax.experimental import pallas as pl
from jax.experimental.pallas import tpu as pltpu
from jax.experimental.pallas import tpu_sc as plsc

_S = 8192
_D = 2048
_B = 4
_NC = 2
_NS = 16
_NW = _NC * _NS
_ROWS_PER_W = _S // _NW
_LENS = [32, 24, 32, 24, 32, 24, 32, 24, 32]
assert sum(_LENS) == _ROWS_PER_W
_OFFS = [sum(_LENS[:i]) for i in range(len(_LENS))]
_NCHUNK = len(_LENS)

_mesh = plsc.VectorSubcoreMesh(core_axis_name="c", subcore_axis_name="s")


@functools.partial(
    pl.kernel,
    mesh=_mesh,
    out_type=jax.ShapeDtypeStruct((_B * _S, _D), jnp.float32),
    scratch_types=[
        pltpu.VMEM((32, _D), jnp.float32),
        pltpu.VMEM((24, _D), jnp.float32),
        pltpu.SemaphoreType.DMA,
        pltpu.SemaphoreType.DMA,
    ],
)
def _bcast_rows(table_hbm, out_hbm, buf_a, buf_b, rsem, wsem):
    wid = lax.axis_index("s") * _NC + lax.axis_index("c")
    base = wid * _ROWS_PER_W
    bufs = [buf_a, buf_b]

    def issue_read(i):
        return pltpu.async_copy(
            table_hbm.at[pl.ds(base + _OFFS[i], _LENS[i])],
            bufs[i % 2],
            rsem,
        )

    def issue_writes(i):
        return [
            pltpu.async_copy(
                bufs[i % 2],
                out_hbm.at[pl.ds(b * _S + base + _OFFS[i], _LENS[i])],
                wsem,
            )
            for b in range(_B)
        ]

    rh = [None] * _NCHUNK
    wh = [None] * _NCHUNK
    rh[0] = issue_read(0)
    rh[1] = issue_read(1)
    for g in range((_NCHUNK + 1) // 2):
        i0, i1 = 2 * g, 2 * g + 1
        rh[i0].wait()
        wh[i0] = issue_writes(i0)
        if i1 < _NCHUNK:
            rh[i1].wait()
            wh[i1] = issue_writes(i1)
        for i in (i0, i1):
            if i + 2 < _NCHUNK:
                for c in wh[i]:
                    c.wait()
                rh[i + 2] = issue_read(i + 2)
    for i in range(max(0, _NCHUNK - 2), _NCHUNK):
        for c in wh[i]:
            c.wait()


def kernel(x, table):
    del x
    out = _bcast_rows(table)
    return out.reshape(_B, _S, _D)


# speedup vs baseline: 1.0180x; 1.0180x over previous
import functools

import jax
import jax.numpy as jnp
from jax import lax
from jax.experimental import pallas as pl
from jax.experimental.pallas import tpu as pltpu
from jax.experimental.pallas import tpu_sc as plsc

_S = 8192
_D = 2048
_B = 4
_NC = 2
_NS = 16
_NW = _NC * _NS
_ROWS_PER_W = _S // _NW
_LENS = [56, 56, 56, 56, 32]
assert sum(_LENS) == _ROWS_PER_W
_OFFS = [sum(_LENS[:i]) for i in range(len(_LENS))]
_NCHUNK = len(_LENS)

_mesh = plsc.VectorSubcoreMesh(core_axis_name="c", subcore_axis_name="s")


@functools.partial(
    pl.kernel,
    mesh=_mesh,
    out_type=jax.ShapeDtypeStruct((_B * _S, _D), jnp.float32),
    scratch_types=[
        pltpu.VMEM((56, _D), jnp.float32),
        pltpu.SemaphoreType.DMA,
        pltpu.SemaphoreType.DMA,
    ],
)
def _bcast_rows(table_hbm, out_hbm, buf, rsem, wsem):
    wid = lax.axis_index("s") * _NC + lax.axis_index("c")
    base = wid * _ROWS_PER_W

    for i in range(_NCHUNK):
        pltpu.async_copy(
            table_hbm.at[pl.ds(base + _OFFS[i], _LENS[i])],
            buf.at[pl.ds(0, _LENS[i])],
            rsem,
        ).wait()
        whs = [
            pltpu.async_copy(
                buf.at[pl.ds(0, _LENS[i])],
                out_hbm.at[pl.ds(b * _S + base + _OFFS[i], _LENS[i])],
                wsem,
            )
            for b in range(_B)
        ]
        for c in whs:
            c.wait()


def kernel(x, table):
    del x
    out = _bcast_rows(table)
    return out.reshape(_B, _S, _D)
